# Initial kernel scaffold; baseline (speedup 1.0000x reference)
import jax
import jax.numpy as jnp
from jax.experimental import pallas as pl


def kernel(hv, edge_index, he, Wm, bm, Wih, Whh, bih, bhh):
    raise NotImplementedError("write your pallas kernel here")


# speedup vs baseline: 3.5892x; 3.5892x over previous
import functools

import jax
import jax.numpy as jnp
from jax import lax
from jax.experimental import pallas as pl
from jax.experimental.pallas import tpu as pltpu
from jax.experimental.pallas import tpu_sc as plsc

_NC = 2
_NS = 16
_CHUNK = 128


def _sc_mesh():
  return plsc.VectorSubcoreMesh(
      core_axis_name="c", subcore_axis_name="s", num_cores=_NC,
      num_subcores=_NS)


def _make_sc_s_pass(npad, d, cpt):
  rows_per_tile = npad // _NS
  zc = rows_per_tile // _CHUNK

  def body(h_hbm, src_hbm, dst_hbm, zrows_hbm, s_out,
           s_sh, src_v, dst_v, rows_v, sem):
    cid = lax.axis_index("c")
    sid = lax.axis_index("s")
    wid = cid * _NS + sid
    row0 = sid * rows_per_tile
    for k in range(zc):
      pltpu.sync_copy(zrows_hbm, s_sh.at[pl.ds(row0 + k * _CHUNK, _CHUNK)])
    plsc.subcore_barrier()

    def step(j, carry):
      base = (wid * cpt + j) * _CHUNK
      pltpu.sync_copy(src_hbm.at[pl.ds(base, _CHUNK)], src_v)
      pltpu.sync_copy(dst_hbm.at[pl.ds(base, _CHUNK)], dst_v)
      pltpu.async_copy(h_hbm.at[src_v], rows_v, sem).wait()
      pltpu.sync_copy(rows_v, s_sh.at[dst_v], add=True)
      return carry

    lax.fori_loop(0, cpt, step, 0)
    plsc.subcore_barrier()
    for k in range(zc):
      r = row0 + k * _CHUNK
      pltpu.sync_copy(s_sh.at[pl.ds(r, _CHUNK)],
                      s_out.at[cid, pl.ds(r, _CHUNK)])

  return pl.kernel(
      body,
      out_type=jax.ShapeDtypeStruct((_NC, npad, d), jnp.float32),
      mesh=_sc_mesh(),
      scratch_types=[
          pltpu.VMEM_SHARED((npad, d), jnp.float32),
          pltpu.VMEM((_CHUNK,), jnp.int32),
          pltpu.VMEM((_CHUNK,), jnp.int32),
          pltpu.VMEM((_CHUNK, d), jnp.float32),
          pltpu.SemaphoreType.DMA,
      ])


def _make_sc_aux_pass(npad, aw, cpt):
  rows_per_tile = npad // _NS
  zc = rows_per_tile // _CHUNK

  def body(heaug_hbm, dst_hbm, zaux_hbm, aux_out,
           aux_sh, dst_v, he_v):
    cid = lax.axis_index("c")
    sid = lax.axis_index("s")
    wid = cid * _NS + sid
    row0 = sid * rows_per_tile
    for k in range(zc):
      pltpu.sync_copy(zaux_hbm, aux_sh.at[pl.ds(row0 + k * _CHUNK, _CHUNK)])
    plsc.subcore_barrier()

    def step(j, carry):
      base = (wid * cpt + j) * _CHUNK
      pltpu.sync_copy(dst_hbm.at[pl.ds(base, _CHUNK)], dst_v)
      pltpu.sync_copy(heaug_hbm.at[pl.ds(base, _CHUNK)], he_v)
      pltpu.sync_copy(he_v, aux_sh.at[dst_v], add=True)
      return carry

    lax.fori_loop(0, cpt, step, 0)
    plsc.subcore_barrier()
    for k in range(zc):
      r = row0 + k * _CHUNK
      pltpu.sync_copy(aux_sh.at[pl.ds(r, _CHUNK)],
                      aux_out.at[cid, pl.ds(r, _CHUNK)])

  return pl.kernel(
      body,
      out_type=jax.ShapeDtypeStruct((_NC, npad, aw), jnp.float32),
      mesh=_sc_mesh(),
      scratch_types=[
          pltpu.VMEM_SHARED((npad, aw), jnp.float32),
          pltpu.VMEM((_CHUNK,), jnp.int32),
          pltpu.VMEM((_CHUNK, aw), jnp.float32),
      ])


def _r32(x):
  return x.astype(jnp.bfloat16).astype(jnp.float32)


def _rne(x):
  u = jax.lax.bitcast_convert_type(x, jnp.uint32)
  lsb = jax.lax.shift_right_logical(u, jnp.uint32(16)) & jnp.uint32(1)
  r = (u + jnp.uint32(0x7FFF) + lsb) & jnp.uint32(0xFFFF0000)
  return jax.lax.bitcast_convert_type(r, jnp.float32)


def _bdot(x, y):
  return jax.lax.dot(x.astype(jnp.bfloat16), y.astype(jnp.bfloat16),
                     preferred_element_type=jnp.float32)


def _fdot(x, y):
  return jax.lax.dot(x, y, preferred_element_type=jnp.float32)


def _split_dot(x, y_r):
  x1 = _rne(x)
  x2 = _rne(x - x1)
  x3 = _rne(x - x1 - x2)
  return _bdot(x1, y_r) + _bdot(x2, y_r) + _bdot(x3, y_r)


def _hwq(x, bn):
  n, d = x.shape

  def body(x_ref, eye_ref, o_ref):
    o_ref[...] = _fdot(x_ref[...], eye_ref[...])

  return pl.pallas_call(
      body,
      grid=(n // bn,),
      in_specs=[pl.BlockSpec((bn, d), lambda i: (i, 0)),
                pl.BlockSpec((d, d), lambda i: (0, 0))],
      out_specs=pl.BlockSpec((bn, d), lambda i: (i, 0)),
      out_shape=jax.ShapeDtypeStruct((n, d), jnp.float32),
  )(x, jnp.eye(d, dtype=jnp.float32))


def _tc_round(h, s0, s1, a0, a1, wsrc_t, wdst_t, whe_t, bm_t,
              wih_t, whh_t, bih_t, bhh_t, bn):
  n, d = h.shape
  ed = whe_t.shape[0]
  g3 = wih_t.shape[1]

  def body(h_ref, s0_ref, s1_ref, a0_ref, a1_ref, wsrc_ref, wdst_ref,
           whe_ref, bm_ref, wih_ref, whh_ref, bih_ref, bhh_ref, out_ref):
    hb = h_ref[...]
    s = s0_ref[...] + s1_ref[...]
    aux = a0_ref[...] + a1_ref[...]
    hesum = aux[:, :ed]
    deg = aux[:, ed:ed + 1]
    a = (_split_dot(s, wsrc_ref[...]) + deg * _fdot(hb, wdst_ref[...])
         + _split_dot(hesum, whe_ref[...]) + deg * bm_ref[...])
    gi = _fdot(a, wih_ref[...]) + bih_ref[...]
    gh = _fdot(hb, whh_ref[...]) + bhh_ref[...]
    r = jax.nn.sigmoid(gi[:, :d] + gh[:, :d])
    z = jax.nn.sigmoid(gi[:, d:2 * d] + gh[:, d:2 * d])
    nn = jnp.tanh(gi[:, 2 * d:] + r * gh[:, 2 * d:])
    out_ref[...] = (1.0 - z) * nn + z * hb

  grid = (n // bn,)
  row_spec = lambda w: pl.BlockSpec((bn, w), lambda i: (i, 0))
  full = lambda shp: pl.BlockSpec(shp, lambda i: (0,) * len(shp))
  return pl.pallas_call(
      body,
      grid=grid,
      in_specs=[row_spec(d), row_spec(d), row_spec(d),
                row_spec(a0.shape[1]), row_spec(a1.shape[1]),
                full((d, d)), full((d, d)), full((ed, d)), full((1, d)),
                full((d, g3)), full((d, g3)), full((1, g3)), full((1, g3))],
      out_specs=row_spec(d),
      out_shape=jax.ShapeDtypeStruct((n, d), jnp.float32),
  )(h, s0, s1, a0, a1, wsrc_t, wdst_t, whe_t, bm_t, wih_t, whh_t,
    bih_t, bhh_t)


def kernel(hv, edge_index, he, Wm, bm, Wih, Whh, bih, bhh):
  n, d = hv.shape
  e = edge_index.shape[1]
  ed = he.shape[1]
  rounds = Wm.shape[0]
  aw = 128

  npad = -(-n // (_NS * _CHUNK)) * (_NS * _CHUNK)
  cpt = -(-e // (_CHUNK * _NC * _NS))
  epad = cpt * _CHUNK * _NC * _NS

  src = edge_index[0].astype(jnp.int32)
  dst = edge_index[1].astype(jnp.int32)
  pad = epad - e
  srcp = jnp.concatenate([src, jnp.zeros((pad,), jnp.int32)])
  dstp = jnp.concatenate([dst, jnp.full((pad,), n, jnp.int32)])
  heaug = _hwq(jnp.concatenate(
      [he, jnp.ones((e, 1), he.dtype),
       jnp.zeros((e, aw - ed - 1), he.dtype)],
      axis=1), bn=1000)
  heaugp = jnp.concatenate([heaug, jnp.zeros((pad, aw), he.dtype)], axis=0)
  zrows = jnp.zeros((_CHUNK, d), jnp.float32)
  zaux = jnp.zeros((_CHUNK, aw), jnp.float32)

  sc_s = _make_sc_s_pass(npad, d, cpt)
  sc_aux = _make_sc_aux_pass(npad, aw, cpt)

  aux_parts = sc_aux(heaugp, dstp, zaux)
  a0, a1 = aux_parts[0, :n], aux_parts[1, :n]
  h = hv
  for t in range(rounds):
    s_parts = sc_s(_hwq(h, bn=1000), srcp, dstp, zrows)
    s0, s1 = s_parts[0, :n], s_parts[1, :n]
    wq = _hwq(Wm[t].T, bn=Wm[t].shape[1])
    wsrc_t = wq[:d]
    wdst_t = Wm[t, :, d:2 * d].T
    whe_t = wq[2 * d:]
    h = _tc_round(h, s0, s1, a0, a1, wsrc_t, wdst_t, whe_t,
                  bm[t].reshape(1, -1), Wih[t].T, Whh[t].T,
                  bih[t].reshape(1, -1), bhh[t].reshape(1, -1), bn=1000)
  return h
